# baseline (speedup 1.0000x reference)
import functools

import jax
import jax.numpy as jnp
from jax import lax
from jax.experimental import pallas as pl
from jax.experimental.pallas import tpu as pltpu
from jax.experimental.pallas import tpu_sc as plsc

_VOCAB = 1_000_000
_D = 32
_L = 24
_B = 16384
_C = 20

_NC = 2
_NS = 16
_NW = _NC * _NS
_BW = _B // _NW
_ER = _VOCAB // 4
_PR = _VOCAB * _L // 128
_FILL = 40.0


def _sc_body(ctx_hbm, tgt_hbm, prow_hbm, poff_hbm, emb_hbm, node_hbm,
             paths_hbm, codes_hbm, lens_hbm, out_hbm,
             tgt_v, prow_v, poff_v, ctx_v, lens_v, cb_v, flat_v, mean_v,
             big, pv, ridx, s3, sem):
  wid = lax.axis_index("s") * _NC + lax.axis_index("c")
  iota = lax.iota(jnp.int32, 16)
  zeros = jnp.zeros((16,), jnp.float32)

  def ld(ref, f):
    return plsc.load_gather(
        ref, [lax.shift_right_logical(f, 7), f & 127])

  def st(ref, f, v):
    plsc.store_scatter(ref, [lax.shift_right_logical(f, 7), f & 127], v)

  blk8 = lax.shift_right_logical(wid, 1) * 8
  lbase = (wid & 1) * 512
  pltpu.sync_copy(tgt_hbm.at[pl.ds(blk8, 8)], tgt_v)
  pltpu.sync_copy(prow_hbm.at[pl.ds(blk8, 8)], prow_v)
  pltpu.sync_copy(poff_hbm.at[pl.ds(blk8, 8)], poff_v)
  pltpu.sync_copy(ctx_hbm.at[pl.ds(wid * 80, 80)], ctx_v)

  j0 = (wid & 1) * 4
  ldescs = [pltpu.async_copy(lens_hbm.at[tgt_v.at[j0 + j]], lens_v.at[j], sem)
            for j in range(4)]
  for d in ldescs:
    d.wait()

  def path_chunk(ck, carry):
    for k in range(8):
      e = k * 16 + iota
      t_loc = ck * 64 + lax.shift_right_logical(e, 1)
      v = ld(prow_v, lbase + t_loc) + (e & 1)
      ridx[0, pl.ds(k * 16, 16)] = jnp.minimum(v, _PR - 1)
    pltpu.async_copy(paths_hbm.at[ridx.at[0]], pv, sem).wait()
    for g in range(4):
      jv = g * 16 + iota
      t_loc = ck * 64 + jv
      po = ld(poff_v, lbase + t_loc)
      rb = lax.shift_left(jv, 1)
      for l in range(_L):
        pos = po + l
        n = plsc.load_gather(
            pv, [rb + lax.shift_right_logical(pos, 7), pos & 127])
        st(flat_v, t_loc * _L + l, n)
    pltpu.async_copy(codes_hbm.at[ridx.at[0]], pv, sem).wait()
    for g in range(4):
      jv = g * 16 + iota
      t_loc = ck * 64 + jv
      po = ld(poff_v, lbase + t_loc)
      rb = lax.shift_left(jv, 1)
      cbacc = jnp.zeros((16,), jnp.int32)
      for l in range(_L):
        pos = po + l
        cbit = plsc.load_gather(
            pv, [rb + lax.shift_right_logical(pos, 7), pos & 127])
        cbacc = cbacc | lax.shift_left(cbit & 1, l)
      st(cb_v, t_loc, cbacc)
    return carry

  lax.fori_loop(0, 8, path_chunk, 0)

  inv_c = jnp.float32(1.0 / _C)

  def ctx_chunk(ck, carry):
    for r in range(3):
      for k in range(8):
        if r * 128 + k * 16 >= 320:
          ridx[r, pl.ds(k * 16, 16)] = jnp.zeros((16,), jnp.int32)
        else:
          cid = ld(ctx_v, ck * 320 + r * 128 + k * 16 + iota)
          ridx[r, pl.ds(k * 16, 16)] = lax.shift_right_logical(cid, 2)
    descs = [pltpu.async_copy(emb_hbm.at[ridx.at[r]],
                              big.at[pl.ds(r * 128, 128)], sem)
             for r in range(3)]
    for d in descs:
      d.wait()
    rows = []
    cols = []
    for c in range(_C):
      e = iota * _C + c
      cid = ld(ctx_v, ck * 320 + e)
      rows.append(e)
      cols.append(lax.shift_left(cid & 3, 5))
    for d_ in range(_D):
      acc = zeros
      for c in range(_C):
        acc = acc + plsc.load_gather(big, [rows[c], cols[c] + d_])
      st(mean_v, ck * 512 + iota * _D + d_, acc * inv_c)
    return carry

  lax.fori_loop(0, 32, ctx_chunk, 0)

  def score_chunk(ck, carry):
    for r in range(3):
      for k in range(8):
        n = ld(flat_v, ck * 384 + r * 128 + k * 16 + iota)
        ridx[r, pl.ds(k * 16, 16)] = lax.shift_right_logical(n, 2)
    descs = [pltpu.async_copy(node_hbm.at[ridx.at[r]],
                              big.at[pl.ds(r * 128, 128)], sem)
             for r in range(3)]
    for d in descs:
      d.wait()
    s_loc = ck * 16 + iota
    lens16 = ld(lens_v, s_loc)
    cb16 = ld(cb_v, s_loc)
    means = [ld(mean_v, ck * 512 + iota * _D + d_) for d_ in range(_D)]
    for l in range(_L):
      n16 = ld(flat_v, s_loc * _L + l)
      colb = lax.shift_left(n16 & 3, 5)
      row = iota * _L + l
      acc = zeros
      for d_ in range(_D):
        acc = acc + means[d_] * plsc.load_gather(big, [row, colb + d_])
      code = lax.shift_right_logical(cb16, l) & 1
      sign = code.astype(jnp.float32) * 2.0 - 1.0
      lv = jnp.full((16,), l, jnp.int32)
      val = jnp.where(lv < lens16, sign * acc, _FILL)
      st(s3, ck * 384 + row, val)
    return carry

  lax.fori_loop(0, 32, score_chunk, 0)
  pltpu.sync_copy(s3, out_hbm.at[pl.ds(wid * 96, 96)])


_sc_scores = functools.partial(
    pl.kernel,
    out_type=jax.ShapeDtypeStruct((_B * _L // 128, 128), jnp.float32),
    mesh=plsc.VectorSubcoreMesh(core_axis_name="c", subcore_axis_name="s"),
    compiler_params=pltpu.CompilerParams(needs_layout_passes=False),
    scratch_types=[
        pltpu.VMEM((8, 128), jnp.int32),
        pltpu.VMEM((8, 128), jnp.int32),
        pltpu.VMEM((8, 128), jnp.int32),
        pltpu.VMEM((80, 128), jnp.int32),
        pltpu.VMEM((4, 128), jnp.int32),
        pltpu.VMEM((4, 128), jnp.int32),
        pltpu.VMEM((96, 128), jnp.int32),
        pltpu.VMEM((128, 128), jnp.float32),
        pltpu.VMEM((384, 128), jnp.float32),
        pltpu.VMEM((128, 128), jnp.int32),
        pltpu.VMEM((3, 128), jnp.int32),
        pltpu.VMEM((96, 128), jnp.float32),
        pltpu.SemaphoreType.DMA,
    ],
)(_sc_body)


def _loss_body(x_ref, o_ref):
  x = x_ref[...]
  loss = jnp.log(1.0 + jnp.exp(-jnp.abs(x))) - jnp.minimum(x, 0.0)
  o_ref[0, 0] = jnp.sum(loss) * jnp.float32(1.0 / _B)


_loss = pl.pallas_call(
    _loss_body,
    out_shape=jax.ShapeDtypeStruct((1, 1), jnp.float32),
    out_specs=pl.BlockSpec(memory_space=pltpu.SMEM),
)


@jax.jit
def _impl(context_words, target_words, input_emb, internal_emb, paths, codes,
          path_lens):
  ctx_flat = context_words.astype(jnp.int32).reshape(_B * _C // 128, 128)
  tgt = target_words.astype(jnp.int32)
  tflat = tgt * _L
  prow = lax.shift_right_logical(tflat, 7).reshape(_B // 128, 128)
  poff = (tflat & 127).reshape(_B // 128, 128)
  emb128 = input_emb.reshape(_ER, 128)
  node128 = jnp.concatenate(
      [internal_emb, jnp.zeros((1, _D), jnp.float32)]).reshape(_ER, 128)
  paths128 = paths.astype(jnp.int32).reshape(_PR, 128)
  codes128 = codes.astype(jnp.int32).reshape(_PR, 128)
  scores = _sc_scores(ctx_flat, tgt.reshape(_B // 128, 128), prow, poff,
                      emb128, node128, paths128, codes128,
                      path_lens.astype(jnp.int32))
  return _loss(scores)[0, 0]


def kernel(context_words, target_words, input_emb, internal_emb, paths, codes,
           path_lens):
  return _impl(context_words, target_words, input_emb, internal_emb, paths,
               codes, path_lens)


# speedup vs baseline: 2.8694x; 2.8694x over previous
import functools

import jax
import jax.numpy as jnp
from jax import lax
from jax.experimental import pallas as pl
from jax.experimental.pallas import tpu as pltpu
from jax.experimental.pallas import tpu_sc as plsc

_VOCAB = 1_000_000
_D = 32
_L = 24
_B = 16384
_C = 20

_NC = 2
_NS = 16
_NW = _NC * _NS
_BW = _B // _NW
_SUB = 32
_NSUB = _BW // _SUB
_CHUNK = 128
_FILL = 40.0
_IDMASK = (1 << 20) - 1
_LENCOL = 24


def _sc_body(ctxi_hbm, tgt_hbm, inemb_hbm, nodemb_hbm, paths_hbm, cbits_hbm,
             lens_hbm, out_hbm, tgt_v, paths_v, cb_v, lens_v, flat_idx,
             ctx_idx, mean_v, ctx_rows, node_rows, scores_v, sem):
  wid = lax.axis_index("s") * _NC + lax.axis_index("c")
  base = wid * _BW
  iota = lax.iota(jnp.int32, 16)
  zeros = jnp.zeros((16,), jnp.float32)

  pltpu.sync_copy(tgt_hbm.at[pl.ds(wid * (_BW // _CHUNK), _BW // _CHUNK)],
                  tgt_v)
  pltpu.sync_copy(
      ctxi_hbm.at[pl.ds(wid * (_BW * _C // _CHUNK), _BW * _C // _CHUNK)],
      ctx_idx)

  descs = []
  for j in range(_BW // _CHUNK):
    idx = tgt_v.at[j]
    descs.append(pltpu.async_copy(
        paths_hbm.at[idx], paths_v.at[pl.ds(j * _CHUNK, _CHUNK)], sem))
    descs.append(pltpu.async_copy(cbits_hbm.at[idx], cb_v.at[j], sem))
    descs.append(pltpu.async_copy(lens_hbm.at[idx], lens_v.at[j], sem))
  for d in descs:
    d.wait()

  def flat_body(r8, carry):
    for j in range(8):
      f = r8 * _CHUNK + j * 16 + iota
      v = plsc.load_gather(paths_v, [f // _L, f % _L])
      flat_idx[r8, pl.ds(j * 16, 16)] = v
    return carry

  lax.fori_loop(0, _BW * _L // _CHUNK, flat_body, 0)

  inv_c = jnp.float32(1.0 / _C)

  def ctx_chunk(sc, carry):
    cdescs = []
    for j in range(_SUB * _C // _CHUNK):
      r0 = sc * (_SUB * _C // _CHUNK) + j
      cdescs.append(pltpu.async_copy(
          inemb_hbm.at[ctx_idx.at[r0]],
          ctx_rows.at[pl.ds(j * _CHUNK, _CHUNK)], sem))
    for d in cdescs:
      d.wait()

    def sample_body(s, c2):
      rbase = s * _C
      acc0 = zeros
      acc1 = zeros
      for c in range(_C):
        acc0 = acc0 + ctx_rows[rbase + c, pl.ds(0, 16)]
        acc1 = acc1 + ctx_rows[rbase + c, pl.ds(16, 16)]
      g = sc * _SUB + s
      mean_v[g, pl.ds(0, 16)] = acc0 * inv_c
      mean_v[g, pl.ds(16, 16)] = acc1 * inv_c
      return c2

    lax.fori_loop(0, _SUB, sample_body, 0)
    return carry

  lax.fori_loop(0, _NSUB, ctx_chunk, 0)

  def node_chunk(sc, carry):
    ndescs = []
    for j in range(_SUB * _L // _CHUNK):
      r0 = sc * (_SUB * _L // _CHUNK) + j
      ndescs.append(pltpu.async_copy(
          nodemb_hbm.at[flat_idx.at[r0]],
          node_rows.at[pl.ds(j * _CHUNK, _CHUNK)], sem))
    for d in ndescs:
      d.wait()

    for blk in range(_SUB // 16):
      s0 = sc * _SUB + blk * 16
      lanes = s0 + iota
      lens_t = plsc.load_gather(lens_v, [lanes // _CHUNK, lanes % _CHUNK])
      cb_t = plsc.load_gather(cb_v, [lanes // _CHUNK, lanes % _CHUNK])
      mean_t = [
          plsc.load_gather(mean_v, [lanes, jnp.full((16,), d_, jnp.int32)])
          for d_ in range(_D)
      ]
      row0 = (blk * 16 + iota) * _L

      def l_body(l, c2, row0=row0, lanes=lanes, lens_t=lens_t, cb_t=cb_t,
                 mean_t=mean_t):
        lv = jnp.full((16,), l, jnp.int32)
        acc = zeros
        for d_ in range(_D):
          nv = plsc.load_gather(node_rows,
                                [row0 + l, jnp.full((16,), d_, jnp.int32)])
          acc = acc + mean_t[d_] * nv
        code = lax.shift_right_logical(cb_t, l) & 1
        sign = code.astype(jnp.float32) * 2.0 - 1.0
        val = jnp.where(lv < lens_t, sign * acc, _FILL)
        plsc.store_scatter(scores_v, [lanes, lv], val)
        return c2

      lax.fori_loop(0, _L, l_body, 0)
    return carry

  lax.fori_loop(0, _NSUB, node_chunk, 0)

  pltpu.sync_copy(scores_v, out_hbm.at[pl.ds(base, _BW)])


_sc_scores = functools.partial(
    pl.kernel,
    out_type=jax.ShapeDtypeStruct((_B, _L), jnp.float32),
    mesh=plsc.VectorSubcoreMesh(core_axis_name="c", subcore_axis_name="s"),
    compiler_params=pltpu.CompilerParams(use_tc_tiling_on_sc=False,
                                         needs_layout_passes=False),
    scratch_types=[
        pltpu.VMEM((_BW // _CHUNK, _CHUNK), jnp.int32),
        pltpu.VMEM((_BW, _L), jnp.int32),
        pltpu.VMEM((_BW // _CHUNK, _CHUNK), jnp.int32),
        pltpu.VMEM((_BW // _CHUNK, _CHUNK), jnp.int32),
        pltpu.VMEM((_BW * _L // _CHUNK, _CHUNK), jnp.int32),
        pltpu.VMEM((_BW * _C // _CHUNK, _CHUNK), jnp.int32),
        pltpu.VMEM((_BW, _D), jnp.float32),
        pltpu.VMEM((_SUB * _C, _D), jnp.float32),
        pltpu.VMEM((_SUB * _L, _D), jnp.float32),
        pltpu.VMEM((_BW, _L), jnp.float32),
        pltpu.SemaphoreType.DMA,
    ],
)(_sc_body)


def _loss_body(x_ref, o_ref):
  x = x_ref[...]
  loss = jnp.log(1.0 + jnp.exp(-jnp.abs(x))) - jnp.minimum(x, 0.0)
  o_ref[0, 0] = jnp.sum(loss) * jnp.float32(1.0 / _B)


_loss = pl.pallas_call(
    _loss_body,
    out_shape=jax.ShapeDtypeStruct((1, 1), jnp.float32),
    out_specs=pl.BlockSpec(memory_space=pltpu.SMEM),
)


@jax.jit
def _impl(context_words, target_words, input_emb, internal_emb, paths, codes,
          path_lens):
  ctx_flat = context_words.astype(jnp.int32).reshape(_B * _C // _CHUNK, _CHUNK)
  tgt = target_words.astype(jnp.int32).reshape(_B // _CHUNK, _CHUNK)
  cbits = jnp.sum(codes.astype(jnp.int32) << jnp.arange(_L, dtype=jnp.int32)[None, :],
                  axis=1)
  scores = _sc_scores(ctx_flat, tgt, input_emb, internal_emb,
                      paths.astype(jnp.int32), cbits,
                      path_lens.astype(jnp.int32))
  loss = _loss(scores.reshape(_B * _L // _CHUNK, _CHUNK))
  return loss[0, 0]


def kernel(context_words, target_words, input_emb, internal_emb, paths, codes,
           path_lens):
  return _impl(context_words, target_words, input_emb, internal_emb, paths,
               codes, path_lens)
